# Initial kernel scaffold; baseline (speedup 1.0000x reference)
import jax
import jax.numpy as jnp
from jax import lax
from jax.experimental import pallas as pl
from jax.experimental.pallas import tpu as pltpu
from jax.experimental.pallas import tpu_sc as plsc

B = 16
L_KEEP = 2048
N = 4096
D = 512

_LANES = 16
_ROWS_PER_WORKER = (B * N) // 32
_CHUNK = 64
_NCHUNK = _ROWS_PER_WORKER // _CHUNK
_MROW = B * (L_KEEP + 1)


def _revert_body(tbl_hbm, idx_hbm, rpm_hbm, pos_hbm, out_hbm,
                 idx_v, rpm_v, gidx_v, gbuf, obuf, sem):
    w = lax.axis_index("s") * 2 + lax.axis_index("c")
    b = w // 2
    half = w % 2
    nstart = half * _ROWS_PER_WORKER

    pltpu.sync_copy(idx_hbm.at[pl.ds(b * N + nstart, _ROWS_PER_WORKER)], idx_v)
    pltpu.sync_copy(rpm_hbm.at[pl.ds(b * L_KEEP, L_KEEP)], rpm_v)

    @pl.when(half == 0)
    def _global_token():
        pltpu.sync_copy(tbl_hbm.at[b * (L_KEEP + 1)], gbuf.at[0])
        pltpu.sync_copy(pos_hbm.at[0], obuf.at[0])
        for g in range(D // _LANES):
            sl = pl.ds(g * _LANES, _LANES)
            obuf[0, sl] = obuf[0, sl] + gbuf[0, sl]
        pltpu.sync_copy(obuf.at[0], out_hbm.at[b * (N + 1)])

    def chunk_body(c, carry):
        base = c * _CHUNK
        for i in range(_CHUNK // _LANES):
            idxg = idx_v[pl.ds(base + i * _LANES, _LANES)]
            inb = idxg < L_KEEP
            idxc = jnp.minimum(idxg, L_KEEP - 1)
            rpmg = plsc.load_gather(rpm_v, [idxc])
            keep = inb & (rpmg == 1)
            grow = jnp.where(keep, b * (L_KEEP + 1) + 1 + idxg, _MROW)
            gidx_v[pl.ds(i * _LANES, _LANES)] = grow
        pltpu.async_copy(tbl_hbm.at[gidx_v], gbuf, sem).wait()
        pltpu.sync_copy(pos_hbm.at[pl.ds(1 + nstart + base, _CHUNK)], obuf)

        def row_body(r, rcarry):
            for g in range(D // _LANES):
                sl = pl.ds(g * _LANES, _LANES)
                plsc.addupdate(obuf.at[r, sl], gbuf[r, sl])
            return rcarry

        lax.fori_loop(0, _CHUNK, row_body, 0)
        pltpu.sync_copy(
            obuf, out_hbm.at[pl.ds(b * (N + 1) + 1 + nstart + base, _CHUNK)])
        return carry

    lax.fori_loop(0, _NCHUNK, chunk_body, 0)


@jax.jit
def kernel(val, mask_token, remain_padding_mask, revert_idx, pos_emb):
    tbl = jnp.concatenate(
        [val.reshape(B * (L_KEEP + 1), D), mask_token.astype(jnp.float32)], axis=0)
    idx_flat = revert_idx.reshape(B * N).astype(jnp.int32)
    rpm_flat = remain_padding_mask.reshape(B * L_KEEP).astype(jnp.int32)
    pos2d = pos_emb.reshape(N + 1, D)

    mesh = plsc.VectorSubcoreMesh(core_axis_name="c", subcore_axis_name="s")
    run = pl.kernel(
        _revert_body,
        out_type=jax.ShapeDtypeStruct((B * (N + 1), D), jnp.float32),
        mesh=mesh,
        scratch_types=[
            pltpu.VMEM((_ROWS_PER_WORKER,), jnp.int32),
            pltpu.VMEM((L_KEEP,), jnp.int32),
            pltpu.VMEM((_CHUNK,), jnp.int32),
            pltpu.VMEM((_CHUNK, D), jnp.float32),
            pltpu.VMEM((_CHUNK, D), jnp.float32),
            pltpu.SemaphoreType.DMA,
        ],
    )
    out = run(tbl, idx_flat, rpm_flat, pos2d)
    return out.reshape(B, N + 1, D)


# speedup vs baseline: 1.1601x; 1.1601x over previous
import jax
import jax.numpy as jnp
from jax import lax
from jax.experimental import pallas as pl
from jax.experimental.pallas import tpu as pltpu
from jax.experimental.pallas import tpu_sc as plsc

B = 16
L_KEEP = 2048
N = 4096
D = 512

_LANES = 16
_HALF = 2048
_CHUNK = 64
_NCHUNK = _HALF // _CHUNK
_MROW = B * (L_KEEP + 1)
_IDXBUF = 2056


def _redirect(idxg, rpm_v, b):
    inb = idxg < L_KEEP
    idxc = jnp.minimum(jnp.maximum(idxg, 0), L_KEEP - 1)
    rpmg = plsc.load_gather(rpm_v, [idxc])
    keep = inb & (rpmg == 1)
    return jnp.where(keep, b * (L_KEEP + 1) + 1 + idxg, _MROW)


def _revert_body(tbl_hbm, idx_hbm, rpm_hbm, pos_hbm, out_hbm,
                 idx_v, rpm_v, gidx_v, gbuf, obuf, sem):
    w = lax.axis_index("s") * 2 + lax.axis_index("c")
    b = w // 2
    half = w % 2
    jstart = half * _HALF
    iota = lax.iota(jnp.int32, _LANES)

    off0 = half * 8 - 1
    src0 = b * N + half * (_HALF - 8)
    pltpu.sync_copy(idx_hbm.at[pl.ds(src0, _IDXBUF)], idx_v)
    pltpu.sync_copy(rpm_hbm.at[pl.ds(b * L_KEEP, L_KEEP)], rpm_v)

    def chunk_body(c, carry):
        base = c * _CHUNK
        for i in range(_CHUNK // _LANES):
            l0 = base + i * _LANES
            ids = iota + (l0 + off0)
            idxg = plsc.load_gather(idx_v, [jnp.maximum(ids, 0)])
            grow = _redirect(idxg, rpm_v, b)
            j_abs = iota + (jstart + l0)
            grow = jnp.where(j_abs == 0, b * (L_KEEP + 1), grow)
            gidx_v[pl.ds(i * _LANES, _LANES)] = grow
        pltpu.async_copy(tbl_hbm.at[gidx_v], gbuf, sem).wait()
        pltpu.sync_copy(pos_hbm.at[pl.ds(jstart + base, _CHUNK)], obuf)

        def row_body(r, rcarry):
            rows = jnp.full((_LANES,), r, jnp.int32)
            for g in range(D // _LANES):
                cols = iota + g * _LANES
                x = plsc.load_gather(gbuf, [rows, cols])
                plsc.addupdate_scatter(obuf, [rows, cols], x)
            return rcarry

        lax.fori_loop(0, _CHUNK, row_body, 0)
        pltpu.sync_copy(obuf, out_hbm.at[b, pl.ds(jstart + base, _CHUNK)])
        return carry

    lax.fori_loop(0, _NCHUNK, chunk_body, 0)

    @pl.when(half == 1)
    def _last_row():
        idxg = plsc.load_gather(idx_v, [jnp.full((_LANES,), _IDXBUF - 1,
                                                 jnp.int32)])
        grow = _redirect(idxg, rpm_v, b)
        gidx_v[pl.ds(0, _LANES)] = grow
        pltpu.async_copy(tbl_hbm.at[gidx_v.at[pl.ds(0, 8)]],
                         gbuf.at[pl.ds(0, 8)], sem).wait()
        pltpu.sync_copy(pos_hbm.at[pl.ds(N, 1)], obuf.at[pl.ds(0, 1)])
        rows = jnp.full((_LANES,), 0, jnp.int32)
        for g in range(D // _LANES):
            cols = iota + g * _LANES
            x = plsc.load_gather(gbuf, [rows, cols])
            plsc.addupdate_scatter(obuf, [rows, cols], x)
        pltpu.sync_copy(obuf.at[pl.ds(0, 1)], out_hbm.at[b, pl.ds(N, 1)])


@jax.jit
def kernel(val, mask_token, remain_padding_mask, revert_idx, pos_emb):
    tbl = jnp.concatenate(
        [val.reshape(B * (L_KEEP + 1), D), mask_token.astype(jnp.float32)],
        axis=0)
    idx_flat = revert_idx.reshape(B * N).astype(jnp.int32)
    rpm_flat = remain_padding_mask.reshape(B * L_KEEP).astype(jnp.int32)
    pos2d = pos_emb.reshape(N + 1, D)

    mesh = plsc.VectorSubcoreMesh(core_axis_name="c", subcore_axis_name="s")
    run = pl.kernel(
        _revert_body,
        out_type=jax.ShapeDtypeStruct((B, N + 1, D), jnp.float32),
        mesh=mesh,
        compiler_params=pltpu.CompilerParams(needs_layout_passes=False),
        scratch_types=[
            pltpu.VMEM((_IDXBUF,), jnp.int32),
            pltpu.VMEM((L_KEEP,), jnp.int32),
            pltpu.VMEM((_CHUNK,), jnp.int32),
            pltpu.VMEM((_CHUNK, D), jnp.float32),
            pltpu.VMEM((_CHUNK, D), jnp.float32),
            pltpu.SemaphoreType.DMA,
        ],
    )
    return run(tbl, idx_flat, rpm_flat, pos2d)


# speedup vs baseline: 1.1617x; 1.0014x over previous
import jax
import jax.numpy as jnp
from jax import lax
from jax.experimental import pallas as pl
from jax.experimental.pallas import tpu as pltpu
from jax.experimental.pallas import tpu_sc as plsc

B = 16
L_KEEP = 2048
N = 4096
D = 512

_LANES = 16
_HALF = 2048
_CHUNK = 64
_NCHUNK = _HALF // _CHUNK
_MROW = B * (L_KEEP + 1)
_IDXBUF = 2056


def _redirect(idxg, rpm_v, b):
    inb = idxg < L_KEEP
    idxc = jnp.minimum(jnp.maximum(idxg, 0), L_KEEP - 1)
    rpmg = plsc.load_gather(rpm_v, [idxc])
    keep = inb & (rpmg == 1)
    return jnp.where(keep, b * (L_KEEP + 1) + 1 + idxg, _MROW)


def _revert_body(tbl_hbm, idx_hbm, rpm_hbm, pos_hbm, out_hbm,
                 idx_v, rpm_v, gidx_v, gbuf, obuf, sem):
    w = lax.axis_index("s") * 2 + lax.axis_index("c")
    b = w // 2
    half = w % 2
    jstart = half * _HALF
    iota = lax.iota(jnp.int32, _LANES)

    off0 = half * 8 - 1
    src0 = b * N + half * (_HALF - 8)
    pltpu.sync_copy(idx_hbm.at[pl.ds(src0, _IDXBUF)], idx_v)
    pltpu.sync_copy(rpm_hbm.at[pl.ds(b * L_KEEP, L_KEEP)], rpm_v)

    def chunk_body(c, carry):
        base = c * _CHUNK
        with jax.named_scope("idxcalc"):
            for i in range(_CHUNK // _LANES):
                l0 = base + i * _LANES
                ids = iota + (l0 + off0)
                idxg = plsc.load_gather(idx_v, [jnp.maximum(ids, 0)])
                grow = _redirect(idxg, rpm_v, b)
                j_abs = iota + (jstart + l0)
                grow = jnp.where(j_abs == 0, b * (L_KEEP + 1), grow)
                gidx_v[pl.ds(i * _LANES, _LANES)] = grow
        with jax.named_scope("gather"):
            pltpu.async_copy(tbl_hbm.at[gidx_v], gbuf, sem).wait()
        with jax.named_scope("poscopy"):
            pltpu.sync_copy(pos_hbm.at[pl.ds(jstart + base, _CHUNK)], obuf)

        def row_body(r, rcarry):
            rows = jnp.full((_LANES,), r, jnp.int32)
            for g in range(D // _LANES):
                cols = iota + g * _LANES
                x = plsc.load_gather(gbuf, [rows, cols])
                plsc.addupdate_scatter(obuf, [rows, cols], x)
            return rcarry

        with jax.named_scope("addloop"):
            lax.fori_loop(0, _CHUNK, row_body, 0)
        with jax.named_scope("outcopy"):
            pltpu.sync_copy(obuf, out_hbm.at[b, pl.ds(jstart + base, _CHUNK)])
        return carry

    lax.fori_loop(0, _NCHUNK, chunk_body, 0)

    @pl.when(half == 1)
    def _last_row():
        idxg = plsc.load_gather(idx_v, [jnp.full((_LANES,), _IDXBUF - 1,
                                                 jnp.int32)])
        grow = _redirect(idxg, rpm_v, b)
        gidx_v[pl.ds(0, _LANES)] = grow
        pltpu.async_copy(tbl_hbm.at[gidx_v.at[pl.ds(0, 8)]],
                         gbuf.at[pl.ds(0, 8)], sem).wait()
        pltpu.sync_copy(pos_hbm.at[pl.ds(N, 1)], obuf.at[pl.ds(0, 1)])
        rows = jnp.full((_LANES,), 0, jnp.int32)
        for g in range(D // _LANES):
            cols = iota + g * _LANES
            x = plsc.load_gather(gbuf, [rows, cols])
            plsc.addupdate_scatter(obuf, [rows, cols], x)
        pltpu.sync_copy(obuf.at[pl.ds(0, 1)], out_hbm.at[b, pl.ds(N, 1)])


@jax.jit
def kernel(val, mask_token, remain_padding_mask, revert_idx, pos_emb):
    tbl = jnp.concatenate(
        [val.reshape(B * (L_KEEP + 1), D), mask_token.astype(jnp.float32)],
        axis=0)
    idx_flat = revert_idx.reshape(B * N).astype(jnp.int32)
    rpm_flat = remain_padding_mask.reshape(B * L_KEEP).astype(jnp.int32)
    pos2d = pos_emb.reshape(N + 1, D)

    mesh = plsc.VectorSubcoreMesh(core_axis_name="c", subcore_axis_name="s")
    run = pl.kernel(
        _revert_body,
        out_type=jax.ShapeDtypeStruct((B, N + 1, D), jnp.float32),
        mesh=mesh,
        compiler_params=pltpu.CompilerParams(needs_layout_passes=False),
        scratch_types=[
            pltpu.VMEM((_IDXBUF,), jnp.int32),
            pltpu.VMEM((L_KEEP,), jnp.int32),
            pltpu.VMEM((_CHUNK,), jnp.int32),
            pltpu.VMEM((_CHUNK, D), jnp.float32),
            pltpu.VMEM((_CHUNK, D), jnp.float32),
            pltpu.SemaphoreType.DMA,
        ],
    )
    return run(tbl, idx_flat, rpm_flat, pos2d)
